# baseline (speedup 1.0000x reference)
import functools

import jax
import jax.numpy as jnp
from jax.experimental import pallas as pl
from jax.experimental.pallas import tpu as pltpu

_N_ADDR = 65536
_D = 1024
_TEMPERATURE = 0.1
_EPS = 1e-8
_BLK = 4096
_GRID = _N_ADDR // _BLK


def _body(q_ref, a_ref, o_ref, acc_ref, den_ref):
    i = pl.program_id(0)

    @pl.when(i == 0)
    def _init():
        acc_ref[...] = jnp.zeros_like(acc_ref)
        den_ref[...] = jnp.zeros_like(den_ref)

    a = a_ref[...]
    q = q_ref[...]
    q_norm = jnp.maximum(jnp.sqrt(jnp.sum(q * q)), _EPS)
    s = jax.lax.dot_general(
        a, q, (((1,), (1,)), ((), ())),
        preferred_element_type=jnp.float32,
        precision=jax.lax.Precision.HIGHEST,
    )
    n2 = jax.lax.dot_general(
        a * a, jnp.ones_like(q), (((1,), (1,)), ((), ())),
        preferred_element_type=jnp.float32,
        precision=jax.lax.Precision.HIGHEST,
    )
    a_norm = jnp.maximum(jnp.sqrt(n2), _EPS)
    cos = s / (a_norm * q_norm)
    w = jnp.exp((cos - 1.0) / _TEMPERATURE)
    acc_ref[...] += jax.lax.dot_general(
        w, a, (((0,), (0,)), ((), ())),
        preferred_element_type=jnp.float32,
        precision=jax.lax.Precision.HIGHEST,
    )
    den_ref[...] += jnp.sum(w)

    @pl.when(i == _GRID - 1)
    def _fin():
        o_ref[...] = acc_ref[...] / den_ref[0, 0]


@jax.jit
def kernel(query_address, addresses):
    out = pl.pallas_call(
        _body,
        grid=(_GRID,),
        in_specs=[
            pl.BlockSpec((1, _D), lambda i: (0, 0)),
            pl.BlockSpec((_BLK, _D), lambda i: (i, 0)),
        ],
        out_specs=pl.BlockSpec((1, _D), lambda i: (0, 0)),
        out_shape=jax.ShapeDtypeStruct((1, _D), jnp.float32),
        scratch_shapes=[
            pltpu.VMEM((1, _D), jnp.float32),
            pltpu.VMEM((1, 1), jnp.float32),
        ],
        compiler_params=pltpu.CompilerParams(
            dimension_semantics=("arbitrary",),
        ),
    )(query_address.reshape(1, _D), addresses)
    return out.reshape(_D)


# speedup vs baseline: 1.8461x; 1.8461x over previous
import functools

import jax
import jax.numpy as jnp
from jax.experimental import pallas as pl
from jax.experimental.pallas import tpu as pltpu

_N_ADDR = 65536
_D = 1024
_TEMPERATURE = 0.1
_EPS = 1e-8
_BLK = 4096
_GRID = _N_ADDR // _BLK


def _body(q_ref, a_ref, o_ref, acc_ref, den_ref):
    i = pl.program_id(0)

    @pl.when(i == 0)
    def _init():
        acc_ref[...] = jnp.zeros_like(acc_ref)
        den_ref[...] = jnp.zeros_like(den_ref)

    a = a_ref[...]
    q = q_ref[...]
    q_norm = jnp.maximum(jnp.sqrt(jnp.sum(q * q)), _EPS)
    s = jax.lax.dot_general(
        a, q, (((1,), (1,)), ((), ())),
        preferred_element_type=jnp.float32,
        precision=jax.lax.Precision.DEFAULT,
    )
    n2 = jax.lax.dot_general(
        a * a, jnp.ones_like(q), (((1,), (1,)), ((), ())),
        preferred_element_type=jnp.float32,
        precision=jax.lax.Precision.DEFAULT,
    )
    a_norm = jnp.maximum(jnp.sqrt(n2), _EPS)
    cos = s / (a_norm * q_norm)
    w = jnp.exp((cos - 1.0) / _TEMPERATURE)
    acc_ref[...] += jax.lax.dot_general(
        w, a, (((0,), (0,)), ((), ())),
        preferred_element_type=jnp.float32,
        precision=jax.lax.Precision.DEFAULT,
    )
    den_ref[...] += jnp.sum(w)

    @pl.when(i == _GRID - 1)
    def _fin():
        o_ref[...] = acc_ref[...] / den_ref[0, 0]


@jax.jit
def kernel(query_address, addresses):
    out = pl.pallas_call(
        _body,
        grid=(_GRID,),
        in_specs=[
            pl.BlockSpec((1, _D), lambda i: (0, 0)),
            pl.BlockSpec((_BLK, _D), lambda i: (i, 0)),
        ],
        out_specs=pl.BlockSpec((1, _D), lambda i: (0, 0)),
        out_shape=jax.ShapeDtypeStruct((1, _D), jnp.float32),
        scratch_shapes=[
            pltpu.VMEM((1, _D), jnp.float32),
            pltpu.VMEM((1, 1), jnp.float32),
        ],
        compiler_params=pltpu.CompilerParams(
            dimension_semantics=("arbitrary",),
        ),
    )(query_address.reshape(1, _D), addresses)
    return out.reshape(_D)
